# baseline (speedup 1.0000x reference)
import jax
import jax.numpy as jnp
from jax import lax
from jax.experimental import pallas as pl
from jax.experimental.pallas import tpu as pltpu

_B, _A, _NBH = 10, 1000, 32
_D = 128
_NG = 25
_NI = 3
_NFB = 3
_CUTOFF = 5.0
_MAXZ = 100

_AP = 1024
_CA = 256
_NC = _AP // _CA
_E = _CA * _NBH
_NGP = 32


def _mega_kernel(az_ref, ehi_ref, elo_ref, p_ref, pc_ref, nbr_ref,
                 iw_ref, ib_ref, win_ref, bin_ref, wh_ref, bh_ref,
                 f2w_ref, f2b_ref, dw_ref, db_ref,
                 xo_ref, xall_ref, y_ref, fc_ref):
    b = pl.program_id(0)
    i = pl.program_id(1)
    c = pl.program_id(2)
    gelu = jax.nn.gelu
    bf16 = jnp.bfloat16

    @pl.when(jnp.logical_and(i == 0, c == 0))
    def _init_x():
        azi = az_ref[0]
        ziot = lax.broadcasted_iota(jnp.int32, (_AP, 128), 1)
        ohz = (ziot == azi).astype(bf16)
        xe = jnp.dot(ohz, ehi_ref[...], preferred_element_type=jnp.float32)
        xe = xe + jnp.dot(ohz, elo_ref[...], preferred_element_type=jnp.float32)
        for cc in range(_NC):
            xall_ref[cc] = xe[cc * _CA:(cc + 1) * _CA]

    @pl.when(c == 0)
    def _compute_y():
        for cc in range(_NC):
            yc = jnp.dot(xall_ref[cc], iw_ref[0],
                         preferred_element_type=jnp.float32)
            y_ref[cc] = (yc + ib_ref[0]).astype(bf16)

    nbr = nbr_ref[0, 0]
    iot = lax.broadcasted_iota(jnp.int32, (_E, _AP), 1)
    oh = (iot == nbr).astype(bf16)

    @pl.when(i == 0)
    def _compute_fij():
        dall = jnp.dot(oh, p_ref[0], preferred_element_type=jnp.float32)
        pc = pc_ref[0].astype(jnp.float32)
        dall = dall - jnp.concatenate([pc] * _NBH, axis=0)
        r2 = jnp.zeros((_E, 1), jnp.float32)
        for cd in range(3):
            dv = dall[:, cd:cd + 1] + dall[:, cd + 4:cd + 5]
            r2 = r2 + dv * dv
        r = jnp.sqrt(r2)
        width = _CUTOFF / (_NG - 1)
        coeff = -0.5 / (width * width)
        offs = (lax.broadcasted_iota(jnp.int32, (_E, _NGP), 1)
                .astype(jnp.float32) * width)
        fc_ref[c, :, 0:_NGP] = jnp.exp(coeff * (r - offs) ** 2).astype(bf16)
        fc_ref[c, :, _NGP:_NGP + 1] = (r <= _CUTOFF).astype(bf16)

    fij = fc_ref[c, :, 0:_NGP]
    w = gelu((jnp.dot(fij, win_ref[0], preferred_element_type=jnp.float32)
              + bin_ref[0]).astype(bf16))
    for j in range(_NFB):
        w = gelu((jnp.dot(w, wh_ref[0, j], preferred_element_type=jnp.float32)
                  + bh_ref[0, j]).astype(bf16))
    w = w * fc_ref[c, :, _NGP:_NGP + 1]
    yj = jnp.dot(oh[:, 0:_CA], y_ref[0], preferred_element_type=jnp.float32)
    for cc in range(1, _NC):
        yj = yj + jnp.dot(oh[:, cc * _CA:(cc + 1) * _CA], y_ref[cc],
                          preferred_element_type=jnp.float32)
    prod = yj * w
    parts = [prod[k * _CA:(k + 1) * _CA, :] for k in range(_NBH)]
    while len(parts) > 1:
        parts = [parts[j] + parts[j + 1] for j in range(0, len(parts), 2)]
    yagg = parts[0]
    t = gelu(jnp.dot(yagg, f2w_ref[0], preferred_element_type=jnp.float32)
             + f2b_ref[0])
    v = jnp.dot(t, dw_ref[0], preferred_element_type=jnp.float32) + db_ref[0]
    xn = xall_ref[c] + v
    xall_ref[c] = xn
    xo_ref[0, 0] = xn


def kernel(atomic_numbers, positions, cell, cell_offset, neighbors,
           neighbor_mask, atom_mask, emb, filt_Win, filt_bin, filt_Wh,
           filt_bh, in2f_W, in2f_b, f2out_W, f2out_b, dense_W, dense_b):
    f32, bf16 = jnp.float32, jnp.bfloat16
    pada = _AP - _A
    az = jnp.pad(atomic_numbers, ((0, 0), (0, pada))).astype(jnp.int32)[..., None]
    pos = jnp.pad(positions, ((0, 0), (0, pada), (0, 0)))
    phi = pos.astype(bf16)
    plo = (pos - phi.astype(f32)).astype(bf16)
    ptab = jnp.concatenate(
        [phi, jnp.zeros((_B, _AP, 1), bf16), plo,
         jnp.zeros((_B, _AP, _D - 7), bf16)], axis=-1)
    nbr = jnp.pad(neighbors, ((0, 0), (0, pada), (0, 0)))
    nbr_k = (nbr.reshape(_B, _NC, _CA, _NBH).transpose(0, 1, 3, 2)
             .reshape(_B, _NC, _E, 1).astype(jnp.int32))
    ehi16 = emb.astype(bf16)
    ehi = jnp.zeros((128, _D), bf16).at[:_MAXZ].set(ehi16)
    elo = jnp.zeros((128, _D), bf16).at[:_MAXZ].set(
        (emb - ehi16.astype(f32)).astype(bf16))
    winp = jnp.pad(filt_Win, ((0, 0), (0, _NGP - _NG), (0, 0))).astype(bf16)
    wh16 = filt_Wh.astype(bf16)

    xf = pl.pallas_call(
        _mega_kernel, grid=(_B, _NI, _NC),
        in_specs=[
            pl.BlockSpec((1, _AP, 1), lambda b, i, c: (b, 0, 0)),
            pl.BlockSpec((128, _D), lambda b, i, c: (0, 0)),
            pl.BlockSpec((128, _D), lambda b, i, c: (0, 0)),
            pl.BlockSpec((1, _AP, _D), lambda b, i, c: (b, 0, 0)),
            pl.BlockSpec((1, _CA, _D), lambda b, i, c: (b, c, 0)),
            pl.BlockSpec((1, 1, _E, 1), lambda b, i, c: (b, c, 0, 0)),
            pl.BlockSpec((1, _D, _D), lambda b, i, c: (i, 0, 0)),
            pl.BlockSpec((1, 1, _D), lambda b, i, c: (i, 0, 0)),
            pl.BlockSpec((1, _NGP, _D), lambda b, i, c: (i, 0, 0)),
            pl.BlockSpec((1, 1, _D), lambda b, i, c: (i, 0, 0)),
            pl.BlockSpec((1, _NFB, _D, _D), lambda b, i, c: (i, 0, 0, 0)),
            pl.BlockSpec((1, _NFB, 1, _D), lambda b, i, c: (i, 0, 0, 0)),
            pl.BlockSpec((1, _D, _D), lambda b, i, c: (i, 0, 0)),
            pl.BlockSpec((1, 1, _D), lambda b, i, c: (i, 0, 0)),
            pl.BlockSpec((1, _D, _D), lambda b, i, c: (i, 0, 0)),
            pl.BlockSpec((1, 1, _D), lambda b, i, c: (i, 0, 0)),
        ],
        out_specs=pl.BlockSpec((1, 1, _CA, _D), lambda b, i, c: (b, c, 0, 0)),
        out_shape=jax.ShapeDtypeStruct((_B, _NC, _CA, _D), f32),
        scratch_shapes=[pltpu.VMEM((_NC, _CA, _D), f32),
                        pltpu.VMEM((_NC, _CA, _D), bf16),
                        pltpu.VMEM((_NC, _E, _NGP + 8), bf16)],
        compiler_params=pltpu.CompilerParams(
            dimension_semantics=("arbitrary", "arbitrary", "arbitrary")),
    )(az, ehi, elo, ptab, ptab, nbr_k,
      in2f_W, in2f_b.reshape(_NI, 1, _D),
      winp, filt_bin.reshape(_NI, 1, _D),
      wh16, filt_bh.reshape(_NI, _NFB, 1, _D),
      f2out_W, f2out_b.reshape(_NI, 1, _D),
      dense_W, dense_b.reshape(_NI, 1, _D))

    return xf.reshape(_B, _AP, _D)[:, :_A, :]


# speedup vs baseline: 1.1681x; 1.1681x over previous
import jax
import jax.numpy as jnp
from jax import lax
from jax.experimental import pallas as pl
from jax.experimental.pallas import tpu as pltpu

_B, _A, _NBH = 10, 1000, 32
_D = 128
_NG = 25
_NI = 3
_NFB = 3
_CUTOFF = 5.0
_MAXZ = 100

_AP = 1024
_CA = 256
_NC = _AP // _CA
_E = _CA * _NBH
_NGP = 32


def _fij_kernel(nbr_ref, p_ref, pc_ref, fij_ref, c_ref):
    bf16 = jnp.bfloat16
    nbrT = nbr_ref[0, 0]
    iot = lax.broadcasted_iota(jnp.int32, (_AP, _E), 0)
    ohT = (iot == nbrT).astype(bf16)
    dallT = jnp.dot(p_ref[0], ohT, preferred_element_type=jnp.float32)
    pcT = pc_ref[0].astype(jnp.float32)
    dallT = dallT - jnp.concatenate([pcT] * _NBH, axis=1)
    dv = dallT[0:3, :] + dallT[4:7, :]
    r2 = (dv[0:1, :] * dv[0:1, :] + dv[1:2, :] * dv[1:2, :]
          + dv[2:3, :] * dv[2:3, :])
    r = jnp.sqrt(r2)
    width = _CUTOFF / (_NG - 1)
    coeff = -0.5 / (width * width)
    offs = (lax.broadcasted_iota(jnp.int32, (_NGP, _E), 0)
            .astype(jnp.float32) * width)
    fij_ref[0, 0] = jnp.exp(coeff * (r - offs) ** 2).astype(bf16)
    c_ref[0, 0] = (r <= _CUTOFF).astype(bf16)


def _mega_kernel(az_ref, ehi_ref, elo_ref, fij_ref, c_ref, nbr_ref,
                 iw_ref, ib_ref, win_ref, bin_ref, wh_ref, bh_ref,
                 f2w_ref, f2b_ref, dw_ref, db_ref,
                 xo_ref, xall_ref, y_ref):
    i = pl.program_id(0)
    b = pl.program_id(1)
    gelu = jax.nn.gelu
    bf16 = jnp.bfloat16

    @pl.when(jnp.logical_and(i == 0, pl.program_id(2) == 0))
    def _init_x():
        azi = az_ref[0]
        ziot = lax.broadcasted_iota(jnp.int32, (_AP, 128), 1)
        ohz = (ziot == azi).astype(bf16)
        xe = jnp.dot(ohz, ehi_ref[...], preferred_element_type=jnp.float32)
        xe = xe + jnp.dot(ohz, elo_ref[...], preferred_element_type=jnp.float32)
        for cc in range(_NC):
            xall_ref[b, cc] = xe[cc * _CA:(cc + 1) * _CA]

    @pl.when(pl.program_id(2) == 0)
    def _compute_y():
        for cc in range(_NC):
            yc = jnp.dot(xall_ref[b, cc], iw_ref[0],
                         preferred_element_type=jnp.float32)
            y_ref[cc] = (yc + ib_ref[0]).astype(bf16)

    fij = fij_ref[0, 0]
    w = gelu((jnp.dot(fij, win_ref[0], preferred_element_type=jnp.float32)
              + bin_ref[0]).astype(bf16))
    for j in range(_NFB):
        w = gelu((jnp.dot(w, wh_ref[0, j], preferred_element_type=jnp.float32)
                  + bh_ref[0, j]).astype(bf16))
    w = w * c_ref[0, 0]
    nbr = nbr_ref[0, 0]
    iot = lax.broadcasted_iota(jnp.int32, (_E, _AP), 1)
    oh = (iot == nbr).astype(bf16)
    yj = jnp.dot(oh[:, 0:_CA], y_ref[0], preferred_element_type=jnp.float32)
    for cc in range(1, _NC):
        yj = yj + jnp.dot(oh[:, cc * _CA:(cc + 1) * _CA], y_ref[cc],
                          preferred_element_type=jnp.float32)
    prod = yj * w
    parts = [prod[k * _CA:(k + 1) * _CA, :] for k in range(_NBH)]
    while len(parts) > 1:
        parts = [parts[j] + parts[j + 1] for j in range(0, len(parts), 2)]
    yagg = parts[0]
    t = gelu(jnp.dot(yagg, f2w_ref[0], preferred_element_type=jnp.float32)
             + f2b_ref[0])
    v = jnp.dot(t, dw_ref[0], preferred_element_type=jnp.float32) + db_ref[0]
    xn = xall_ref[b, pl.program_id(2)] + v
    xall_ref[b, pl.program_id(2)] = xn
    xo_ref[0, 0] = xn


def _full(shape):
    return pl.BlockSpec(shape, lambda *_: tuple(0 for _ in shape))


def kernel(atomic_numbers, positions, cell, cell_offset, neighbors,
           neighbor_mask, atom_mask, emb, filt_Win, filt_bin, filt_Wh,
           filt_bh, in2f_W, in2f_b, f2out_W, f2out_b, dense_W, dense_b):
    f32, bf16 = jnp.float32, jnp.bfloat16
    pada = _AP - _A
    az = jnp.pad(atomic_numbers, ((0, 0), (0, pada))).astype(jnp.int32)[..., None]
    pos = jnp.pad(positions, ((0, 0), (0, pada), (0, 0)))
    phi = pos.astype(bf16)
    plo = (pos - phi.astype(f32)).astype(bf16)
    ptabT = jnp.concatenate(
        [phi.transpose(0, 2, 1), jnp.zeros((_B, 1, _AP), bf16),
         plo.transpose(0, 2, 1), jnp.zeros((_B, 1, _AP), bf16)],
        axis=1)
    nbr = jnp.pad(neighbors, ((0, 0), (0, pada), (0, 0)))
    nbr_km = (nbr.reshape(_B, _NC, _CA, _NBH).transpose(0, 1, 3, 2)
              .reshape(_B, _NC, _E).astype(jnp.int32))
    nbr_k = nbr_km.reshape(_B, _NC, _E, 1)
    nbr_kT = nbr_km.reshape(_B, _NC, 1, _E)
    ehi16 = emb.astype(bf16)
    ehi = jnp.zeros((128, _D), bf16).at[:_MAXZ].set(ehi16)
    elo = jnp.zeros((128, _D), bf16).at[:_MAXZ].set(
        (emb - ehi16.astype(f32)).astype(bf16))
    winp = jnp.pad(filt_Win, ((0, 0), (0, _NGP - _NG), (0, 0))).astype(bf16)
    wh16 = filt_Wh.astype(bf16)

    fijT, cmaskT = pl.pallas_call(
        _fij_kernel, grid=(_B, _NC),
        in_specs=[pl.BlockSpec((1, 1, 1, _E), lambda b, c: (b, c, 0, 0)),
                  pl.BlockSpec((1, 8, _AP), lambda b, c: (b, 0, 0)),
                  pl.BlockSpec((1, 8, _CA), lambda b, c: (b, 0, c))],
        out_specs=[pl.BlockSpec((1, 1, _NGP, _E), lambda b, c: (b, c, 0, 0)),
                   pl.BlockSpec((1, 1, 1, _E), lambda b, c: (b, c, 0, 0))],
        out_shape=[jax.ShapeDtypeStruct((_B, _NC, _NGP, _E), bf16),
                   jax.ShapeDtypeStruct((_B, _NC, 1, _E), bf16)],
    )(nbr_kT, ptabT, ptabT)
    fij = jnp.swapaxes(fijT, 2, 3)
    cmask = cmaskT.reshape(_B, _NC, _E, 1)

    xf = pl.pallas_call(
        _mega_kernel, grid=(_NI, _B, _NC),
        in_specs=[
            pl.BlockSpec((1, _AP, 1), lambda i, b, c: (b, 0, 0)),
            pl.BlockSpec((128, _D), lambda i, b, c: (0, 0)),
            pl.BlockSpec((128, _D), lambda i, b, c: (0, 0)),
            pl.BlockSpec((1, 1, _E, _NGP), lambda i, b, c: (b, c, 0, 0)),
            pl.BlockSpec((1, 1, _E, 1), lambda i, b, c: (b, c, 0, 0)),
            pl.BlockSpec((1, 1, _E, 1), lambda i, b, c: (b, c, 0, 0)),
            pl.BlockSpec((1, _D, _D), lambda i, b, c: (i, 0, 0)),
            pl.BlockSpec((1, 1, _D), lambda i, b, c: (i, 0, 0)),
            pl.BlockSpec((1, _NGP, _D), lambda i, b, c: (i, 0, 0)),
            pl.BlockSpec((1, 1, _D), lambda i, b, c: (i, 0, 0)),
            pl.BlockSpec((1, _NFB, _D, _D), lambda i, b, c: (i, 0, 0, 0)),
            pl.BlockSpec((1, _NFB, 1, _D), lambda i, b, c: (i, 0, 0, 0)),
            pl.BlockSpec((1, _D, _D), lambda i, b, c: (i, 0, 0)),
            pl.BlockSpec((1, 1, _D), lambda i, b, c: (i, 0, 0)),
            pl.BlockSpec((1, _D, _D), lambda i, b, c: (i, 0, 0)),
            pl.BlockSpec((1, 1, _D), lambda i, b, c: (i, 0, 0)),
        ],
        out_specs=pl.BlockSpec((1, 1, _CA, _D), lambda i, b, c: (b, c, 0, 0)),
        out_shape=jax.ShapeDtypeStruct((_B, _NC, _CA, _D), f32),
        scratch_shapes=[pltpu.VMEM((_B, _NC, _CA, _D), f32),
                        pltpu.VMEM((_NC, _CA, _D), bf16)],
        compiler_params=pltpu.CompilerParams(
            dimension_semantics=("arbitrary", "arbitrary", "arbitrary")),
    )(az, ehi, elo, fij, cmask, nbr_k,
      in2f_W, in2f_b.reshape(_NI, 1, _D),
      winp, filt_bin.reshape(_NI, 1, _D),
      wh16, filt_bh.reshape(_NI, _NFB, 1, _D),
      f2out_W, f2out_b.reshape(_NI, 1, _D),
      dense_W, dense_b.reshape(_NI, 1, _D))

    return xf.reshape(_B, _AP, _D)[:, :_A, :]


# speedup vs baseline: 1.4102x; 1.2073x over previous
import jax
import jax.numpy as jnp
from jax import lax
from jax.experimental import pallas as pl
from jax.experimental.pallas import tpu as pltpu

_B, _A, _NBH = 10, 1000, 32
_D = 128
_NG = 25
_NI = 3
_NFB = 3
_CUTOFF = 5.0
_MAXZ = 100

_AP = 1024
_CA = 256
_NC = _AP // _CA
_E = _CA * _NBH
_NGP = 32


def _fij_kernel(nbr_ref, p_ref, pc_ref, fij_ref, c_ref):
    bf16 = jnp.bfloat16
    nbrT = nbr_ref[0, 0]
    iot = lax.broadcasted_iota(jnp.int32, (_AP, _E), 0)
    ohT = (iot == nbrT).astype(bf16)
    dallT = jnp.dot(p_ref[0], ohT, preferred_element_type=jnp.float32)
    pcT = pc_ref[0].astype(jnp.float32)
    dallT = dallT - jnp.concatenate([pcT] * _NBH, axis=1)
    dv = dallT[0:3, :] + dallT[4:7, :]
    r2 = (dv[0:1, :] * dv[0:1, :] + dv[1:2, :] * dv[1:2, :]
          + dv[2:3, :] * dv[2:3, :])
    r = jnp.sqrt(r2)
    width = _CUTOFF / (_NG - 1)
    coeff = -0.5 / (width * width)
    offs = (lax.broadcasted_iota(jnp.int32, (_NGP, _E), 0)
            .astype(jnp.float32) * width)
    fij_ref[0, 0] = jnp.exp(coeff * (r - offs) ** 2).astype(bf16)
    c_ref[0, 0] = jnp.where(r <= _CUTOFF, nbrT, -1)


def _mega_kernel(az_ref, ehi_ref, elo_ref, fij_ref, nbr_ref,
                 iw_ref, ib_ref, win_ref, bin_ref, wh_ref, bh_ref,
                 f2w_ref, f2b_ref, dw_ref, db_ref,
                 xo_ref, xall_ref, y_ref):
    i = pl.program_id(0)
    b = pl.program_id(1)
    gelu = jax.nn.gelu
    bf16 = jnp.bfloat16

    @pl.when(jnp.logical_and(i == 0, pl.program_id(2) == 0))
    def _init_x():
        azi = az_ref[0]
        ziot = lax.broadcasted_iota(jnp.int32, (_AP, 128), 1)
        ohz = (ziot == azi).astype(bf16)
        xe = jnp.dot(ohz, ehi_ref[...], preferred_element_type=jnp.float32)
        xe = xe + jnp.dot(ohz, elo_ref[...], preferred_element_type=jnp.float32)
        for cc in range(_NC):
            xall_ref[b, cc] = xe[cc * _CA:(cc + 1) * _CA]

    @pl.when(pl.program_id(2) == 0)
    def _compute_y():
        for cc in range(_NC):
            yc = jnp.dot(xall_ref[b, cc], iw_ref[0],
                         preferred_element_type=jnp.float32)
            y_ref[cc] = (yc + ib_ref[0]).astype(bf16)

    fijT = fij_ref[0, 0]
    w = gelu((lax.dot_general(fijT, win_ref[0], (((0,), (0,)), ((), ())),
                              preferred_element_type=jnp.float32)
              + bin_ref[0]).astype(bf16))
    for j in range(_NFB):
        w = gelu((jnp.dot(w, wh_ref[0, j], preferred_element_type=jnp.float32)
                  + bh_ref[0, j]).astype(bf16))
    nbr = nbr_ref[0, 0]
    iot = lax.broadcasted_iota(jnp.int32, (_E, _AP), 1)
    oh = (iot == nbr).astype(bf16)
    yj = jnp.dot(oh[:, 0:_CA], y_ref[0], preferred_element_type=jnp.float32)
    for cc in range(1, _NC):
        yj = yj + jnp.dot(oh[:, cc * _CA:(cc + 1) * _CA], y_ref[cc],
                          preferred_element_type=jnp.float32)
    prod = yj * w
    parts = [prod[k * _CA:(k + 1) * _CA, :] for k in range(_NBH)]
    while len(parts) > 1:
        parts = [parts[j] + parts[j + 1] for j in range(0, len(parts), 2)]
    yagg = parts[0]
    t = gelu(jnp.dot(yagg, f2w_ref[0], preferred_element_type=jnp.float32)
             + f2b_ref[0])
    v = jnp.dot(t, dw_ref[0], preferred_element_type=jnp.float32) + db_ref[0]
    xn = xall_ref[b, pl.program_id(2)] + v
    xall_ref[b, pl.program_id(2)] = xn
    xo_ref[0, 0] = xn


def _full(shape):
    return pl.BlockSpec(shape, lambda *_: tuple(0 for _ in shape))


def kernel(atomic_numbers, positions, cell, cell_offset, neighbors,
           neighbor_mask, atom_mask, emb, filt_Win, filt_bin, filt_Wh,
           filt_bh, in2f_W, in2f_b, f2out_W, f2out_b, dense_W, dense_b):
    f32, bf16 = jnp.float32, jnp.bfloat16
    pada = _AP - _A
    az = jnp.pad(atomic_numbers, ((0, 0), (0, pada))).astype(jnp.int32)[..., None]
    pos = jnp.pad(positions, ((0, 0), (0, pada), (0, 0)))
    phi = pos.astype(bf16)
    plo = (pos - phi.astype(f32)).astype(bf16)
    ptabT = jnp.concatenate(
        [phi.transpose(0, 2, 1), jnp.zeros((_B, 1, _AP), bf16),
         plo.transpose(0, 2, 1), jnp.zeros((_B, 1, _AP), bf16)],
        axis=1)
    nbr = jnp.pad(neighbors, ((0, 0), (0, pada), (0, 0)))
    nbr_km = (nbr.reshape(_B, _NC, _CA, _NBH).transpose(0, 1, 3, 2)
              .reshape(_B, _NC, _E).astype(jnp.int32))
    nbr_k = nbr_km.reshape(_B, _NC, _E, 1)
    nbr_kT = nbr_km.reshape(_B, _NC, 1, _E)
    ehi16 = emb.astype(bf16)
    ehi = jnp.zeros((128, _D), bf16).at[:_MAXZ].set(ehi16)
    elo = jnp.zeros((128, _D), bf16).at[:_MAXZ].set(
        (emb - ehi16.astype(f32)).astype(bf16))
    winp = jnp.pad(filt_Win, ((0, 0), (0, _NGP - _NG), (0, 0))).astype(bf16)
    wh16 = filt_Wh.astype(bf16)

    fijT, cmaskT = pl.pallas_call(
        _fij_kernel, grid=(_B, _NC),
        in_specs=[pl.BlockSpec((1, 1, 1, _E), lambda b, c: (b, c, 0, 0)),
                  pl.BlockSpec((1, 8, _AP), lambda b, c: (b, 0, 0)),
                  pl.BlockSpec((1, 8, _CA), lambda b, c: (b, 0, c))],
        out_specs=[pl.BlockSpec((1, 1, _NGP, _E), lambda b, c: (b, c, 0, 0)),
                   pl.BlockSpec((1, 1, 1, _E), lambda b, c: (b, c, 0, 0))],
        out_shape=[jax.ShapeDtypeStruct((_B, _NC, _NGP, _E), bf16),
                   jax.ShapeDtypeStruct((_B, _NC, 1, _E), jnp.int32)],
    )(nbr_kT, ptabT, ptabT)
    nbr_eff = cmaskT.reshape(_B, _NC, _E, 1)

    xf = pl.pallas_call(
        _mega_kernel, grid=(_NI, _B, _NC),
        in_specs=[
            pl.BlockSpec((1, _AP, 1), lambda i, b, c: (b, 0, 0)),
            pl.BlockSpec((128, _D), lambda i, b, c: (0, 0)),
            pl.BlockSpec((128, _D), lambda i, b, c: (0, 0)),
            pl.BlockSpec((1, 1, _NGP, _E), lambda i, b, c: (b, c, 0, 0)),
            pl.BlockSpec((1, 1, _E, 1), lambda i, b, c: (b, c, 0, 0)),
            pl.BlockSpec((1, _D, _D), lambda i, b, c: (i, 0, 0)),
            pl.BlockSpec((1, 1, _D), lambda i, b, c: (i, 0, 0)),
            pl.BlockSpec((1, _NGP, _D), lambda i, b, c: (i, 0, 0)),
            pl.BlockSpec((1, 1, _D), lambda i, b, c: (i, 0, 0)),
            pl.BlockSpec((1, _NFB, _D, _D), lambda i, b, c: (i, 0, 0, 0)),
            pl.BlockSpec((1, _NFB, 1, _D), lambda i, b, c: (i, 0, 0, 0)),
            pl.BlockSpec((1, _D, _D), lambda i, b, c: (i, 0, 0)),
            pl.BlockSpec((1, 1, _D), lambda i, b, c: (i, 0, 0)),
            pl.BlockSpec((1, _D, _D), lambda i, b, c: (i, 0, 0)),
            pl.BlockSpec((1, 1, _D), lambda i, b, c: (i, 0, 0)),
        ],
        out_specs=pl.BlockSpec((1, 1, _CA, _D), lambda i, b, c: (b, c, 0, 0)),
        out_shape=jax.ShapeDtypeStruct((_B, _NC, _CA, _D), f32),
        scratch_shapes=[pltpu.VMEM((_B, _NC, _CA, _D), f32),
                        pltpu.VMEM((_NC, _CA, _D), bf16)],
        compiler_params=pltpu.CompilerParams(
            dimension_semantics=("arbitrary", "arbitrary", "arbitrary")),
    )(az, ehi, elo, fijT, nbr_eff,
      in2f_W, in2f_b.reshape(_NI, 1, _D),
      winp, filt_bin.reshape(_NI, 1, _D),
      wh16, filt_bh.reshape(_NI, _NFB, 1, _D),
      f2out_W, f2out_b.reshape(_NI, 1, _D),
      dense_W, dense_b.reshape(_NI, 1, _D))

    return xf.reshape(_B, _AP, _D)[:, :_A, :]


# speedup vs baseline: 1.4112x; 1.0007x over previous
import jax
import jax.numpy as jnp
from jax import lax
from jax.experimental import pallas as pl
from jax.experimental.pallas import tpu as pltpu

_B, _A, _NBH = 10, 1000, 32
_D = 128
_NG = 25
_NI = 3
_NFB = 3
_CUTOFF = 5.0
_MAXZ = 100

_AP = 1024
_CA = 256
_NC = _AP // _CA
_E = _CA * _NBH
_NGP = 32


def _fij_kernel(nbr_ref, p_ref, pc_ref, fij_ref, c_ref):
    bf16 = jnp.bfloat16
    nbrT = nbr_ref[0, 0]
    iot = lax.broadcasted_iota(jnp.int32, (_AP, _E), 0)
    ohT = (iot == nbrT).astype(bf16)
    dallT = jnp.dot(p_ref[0], ohT, preferred_element_type=jnp.float32)
    pcT = pc_ref[0].astype(jnp.float32)
    dallT = dallT - jnp.concatenate([pcT] * _NBH, axis=1)
    dv = dallT[0:3, :] + dallT[4:7, :]
    r2 = (dv[0:1, :] * dv[0:1, :] + dv[1:2, :] * dv[1:2, :]
          + dv[2:3, :] * dv[2:3, :])
    r = jnp.sqrt(r2)
    width = _CUTOFF / (_NG - 1)
    coeff = -0.5 / (width * width)
    offs = (lax.broadcasted_iota(jnp.int32, (_NGP, _E), 0)
            .astype(jnp.float32) * width)
    fij_ref[0, 0] = jnp.exp(coeff * (r - offs) ** 2).astype(bf16)
    c_ref[0, 0] = jnp.where(r <= _CUTOFF, nbrT, -1)


def _mega_kernel(az_ref, ehi_ref, elo_ref, fij_ref, nbr_ref,
                 iw_ref, ib_ref, win_ref, bin_ref, wh_ref, bh_ref,
                 f2w_ref, f2b_ref, dw_ref, db_ref,
                 xo_ref, xall_ref, y_ref):
    i = pl.program_id(0)
    b = pl.program_id(1)
    gelu = jax.nn.gelu
    bf16 = jnp.bfloat16

    @pl.when(jnp.logical_and(i == 0, pl.program_id(2) == 0))
    def _init_x():
        azi = az_ref[0]
        ziot = lax.broadcasted_iota(jnp.int32, (_AP, 128), 1)
        ohz = (ziot == azi).astype(bf16)
        xe = jnp.dot(ohz, ehi_ref[...], preferred_element_type=jnp.float32)
        xe = xe + jnp.dot(ohz, elo_ref[...], preferred_element_type=jnp.float32)
        for cc in range(_NC):
            xall_ref[b, cc] = xe[cc * _CA:(cc + 1) * _CA]

    @pl.when(pl.program_id(2) == 0)
    def _compute_y():
        for cc in range(_NC):
            yc = jnp.dot(xall_ref[b, cc], iw_ref[0],
                         preferred_element_type=jnp.float32)
            y_ref[cc * _CA:(cc + 1) * _CA] = (yc + ib_ref[0]).astype(bf16)

    fijT = fij_ref[0, 0]
    w = gelu((lax.dot_general(fijT, win_ref[0], (((0,), (0,)), ((), ())),
                              preferred_element_type=jnp.float32)
              + bin_ref[0]).astype(bf16))
    for j in range(_NFB):
        w = gelu((jnp.dot(w, wh_ref[0, j], preferred_element_type=jnp.float32)
                  + bh_ref[0, j]).astype(bf16))
    nbr = nbr_ref[0, 0]
    iot = lax.broadcasted_iota(jnp.int32, (_E, _AP), 1)
    oh = (iot == nbr).astype(bf16)
    yj = jnp.dot(oh, y_ref[...], preferred_element_type=jnp.float32)
    prod = yj * w
    parts = [prod[k * _CA:(k + 1) * _CA, :] for k in range(_NBH)]
    while len(parts) > 1:
        parts = [parts[j] + parts[j + 1] for j in range(0, len(parts), 2)]
    yagg = parts[0]
    t = gelu(jnp.dot(yagg, f2w_ref[0], preferred_element_type=jnp.float32)
             + f2b_ref[0])
    v = jnp.dot(t, dw_ref[0], preferred_element_type=jnp.float32) + db_ref[0]
    xn = xall_ref[b, pl.program_id(2)] + v
    xall_ref[b, pl.program_id(2)] = xn
    xo_ref[0, 0] = xn


def _full(shape):
    return pl.BlockSpec(shape, lambda *_: tuple(0 for _ in shape))


def kernel(atomic_numbers, positions, cell, cell_offset, neighbors,
           neighbor_mask, atom_mask, emb, filt_Win, filt_bin, filt_Wh,
           filt_bh, in2f_W, in2f_b, f2out_W, f2out_b, dense_W, dense_b):
    f32, bf16 = jnp.float32, jnp.bfloat16
    pada = _AP - _A
    az = jnp.pad(atomic_numbers, ((0, 0), (0, pada))).astype(jnp.int32)[..., None]
    pos = jnp.pad(positions, ((0, 0), (0, pada), (0, 0)))
    phi = pos.astype(bf16)
    plo = (pos - phi.astype(f32)).astype(bf16)
    ptabT = jnp.concatenate(
        [phi.transpose(0, 2, 1), jnp.zeros((_B, 1, _AP), bf16),
         plo.transpose(0, 2, 1), jnp.zeros((_B, 1, _AP), bf16)],
        axis=1)
    nbr = jnp.pad(neighbors, ((0, 0), (0, pada), (0, 0)))
    nbr_km = (nbr.reshape(_B, _NC, _CA, _NBH).transpose(0, 1, 3, 2)
              .reshape(_B, _NC, _E).astype(jnp.int32))
    nbr_k = nbr_km.reshape(_B, _NC, _E, 1)
    nbr_kT = nbr_km.reshape(_B, _NC, 1, _E)
    ehi16 = emb.astype(bf16)
    ehi = jnp.zeros((128, _D), bf16).at[:_MAXZ].set(ehi16)
    elo = jnp.zeros((128, _D), bf16).at[:_MAXZ].set(
        (emb - ehi16.astype(f32)).astype(bf16))
    winp = jnp.pad(filt_Win, ((0, 0), (0, _NGP - _NG), (0, 0))).astype(bf16)
    wh16 = filt_Wh.astype(bf16)

    fijT, cmaskT = pl.pallas_call(
        _fij_kernel, grid=(_B, _NC),
        in_specs=[pl.BlockSpec((1, 1, 1, _E), lambda b, c: (b, c, 0, 0)),
                  pl.BlockSpec((1, 8, _AP), lambda b, c: (b, 0, 0)),
                  pl.BlockSpec((1, 8, _CA), lambda b, c: (b, 0, c))],
        out_specs=[pl.BlockSpec((1, 1, _NGP, _E), lambda b, c: (b, c, 0, 0)),
                   pl.BlockSpec((1, 1, 1, _E), lambda b, c: (b, c, 0, 0))],
        out_shape=[jax.ShapeDtypeStruct((_B, _NC, _NGP, _E), bf16),
                   jax.ShapeDtypeStruct((_B, _NC, 1, _E), jnp.int32)],
    )(nbr_kT, ptabT, ptabT)
    nbr_eff = cmaskT.reshape(_B, _NC, _E, 1)

    xf = pl.pallas_call(
        _mega_kernel, grid=(_NI, _B, _NC),
        in_specs=[
            pl.BlockSpec((1, _AP, 1), lambda i, b, c: (b, 0, 0)),
            pl.BlockSpec((128, _D), lambda i, b, c: (0, 0)),
            pl.BlockSpec((128, _D), lambda i, b, c: (0, 0)),
            pl.BlockSpec((1, 1, _NGP, _E), lambda i, b, c: (b, c, 0, 0)),
            pl.BlockSpec((1, 1, _E, 1), lambda i, b, c: (b, c, 0, 0)),
            pl.BlockSpec((1, _D, _D), lambda i, b, c: (i, 0, 0)),
            pl.BlockSpec((1, 1, _D), lambda i, b, c: (i, 0, 0)),
            pl.BlockSpec((1, _NGP, _D), lambda i, b, c: (i, 0, 0)),
            pl.BlockSpec((1, 1, _D), lambda i, b, c: (i, 0, 0)),
            pl.BlockSpec((1, _NFB, _D, _D), lambda i, b, c: (i, 0, 0, 0)),
            pl.BlockSpec((1, _NFB, 1, _D), lambda i, b, c: (i, 0, 0, 0)),
            pl.BlockSpec((1, _D, _D), lambda i, b, c: (i, 0, 0)),
            pl.BlockSpec((1, 1, _D), lambda i, b, c: (i, 0, 0)),
            pl.BlockSpec((1, _D, _D), lambda i, b, c: (i, 0, 0)),
            pl.BlockSpec((1, 1, _D), lambda i, b, c: (i, 0, 0)),
        ],
        out_specs=pl.BlockSpec((1, 1, _CA, _D), lambda i, b, c: (b, c, 0, 0)),
        out_shape=jax.ShapeDtypeStruct((_B, _NC, _CA, _D), f32),
        scratch_shapes=[pltpu.VMEM((_B, _NC, _CA, _D), f32),
                        pltpu.VMEM((_AP, _D), bf16)],
        compiler_params=pltpu.CompilerParams(
            dimension_semantics=("arbitrary", "arbitrary", "arbitrary")),
    )(az, ehi, elo, fijT, nbr_eff,
      in2f_W, in2f_b.reshape(_NI, 1, _D),
      winp, filt_bin.reshape(_NI, 1, _D),
      wh16, filt_bh.reshape(_NI, _NFB, 1, _D),
      f2out_W, f2out_b.reshape(_NI, 1, _D),
      dense_W, dense_b.reshape(_NI, 1, _D))

    return xf.reshape(_B, _AP, _D)[:, :_A, :]


# speedup vs baseline: 1.4665x; 1.0392x over previous
import jax
import jax.numpy as jnp
from jax import lax
from jax.experimental import pallas as pl
from jax.experimental.pallas import tpu as pltpu

_B, _A, _NBH = 10, 1000, 32
_D = 128
_NG = 25
_NI = 3
_NFB = 3
_CUTOFF = 5.0
_MAXZ = 100

_AP = 1024
_CA = 512
_NC = _AP // _CA
_E = _CA * _NBH
_NGP = 32


def _fij_kernel(nbr_ref, p_ref, pc_ref, fij_ref, c_ref):
    bf16 = jnp.bfloat16
    nbrT = nbr_ref[0, 0]
    iot = lax.broadcasted_iota(jnp.int32, (_AP, _E), 0)
    ohT = (iot == nbrT).astype(bf16)
    dallT = jnp.dot(p_ref[0], ohT, preferred_element_type=jnp.float32)
    pcT = pc_ref[0].astype(jnp.float32)
    dallT = dallT - jnp.concatenate([pcT] * _NBH, axis=1)
    dv = dallT[0:3, :] + dallT[4:7, :]
    r2 = (dv[0:1, :] * dv[0:1, :] + dv[1:2, :] * dv[1:2, :]
          + dv[2:3, :] * dv[2:3, :])
    r = jnp.sqrt(r2)
    width = _CUTOFF / (_NG - 1)
    coeff = -0.5 / (width * width)
    offs = (lax.broadcasted_iota(jnp.int32, (_NGP, _E), 0)
            .astype(jnp.float32) * width)
    fij_ref[0, 0] = jnp.exp(coeff * (r - offs) ** 2).astype(bf16)
    c_ref[0, 0] = jnp.where(r <= _CUTOFF, nbrT, -1)


def _mega_kernel(az_ref, ehi_ref, elo_ref, fij_ref, nbr_ref,
                 iw_ref, ib_ref, win_ref, bin_ref, wh_ref, bh_ref,
                 f2w_ref, f2b_ref, dw_ref, db_ref,
                 xo_ref, xall_ref, y_ref):
    i = pl.program_id(0)
    b = pl.program_id(1)
    gelu = jax.nn.gelu
    bf16 = jnp.bfloat16

    @pl.when(jnp.logical_and(i == 0, pl.program_id(2) == 0))
    def _init_x():
        azi = az_ref[0]
        ziot = lax.broadcasted_iota(jnp.int32, (_AP, 128), 1)
        ohz = (ziot == azi).astype(bf16)
        xe = jnp.dot(ohz, ehi_ref[...], preferred_element_type=jnp.float32)
        xe = xe + jnp.dot(ohz, elo_ref[...], preferred_element_type=jnp.float32)
        for cc in range(_NC):
            xall_ref[b, cc] = xe[cc * _CA:(cc + 1) * _CA]

    @pl.when(pl.program_id(2) == 0)
    def _compute_y():
        for cc in range(_NC):
            yc = jnp.dot(xall_ref[b, cc], iw_ref[0],
                         preferred_element_type=jnp.float32)
            y_ref[cc * _CA:(cc + 1) * _CA] = (yc + ib_ref[0]).astype(bf16)

    fijT = fij_ref[0, 0]
    w = gelu((lax.dot_general(fijT, win_ref[0], (((0,), (0,)), ((), ())),
                              preferred_element_type=jnp.float32)
              + bin_ref[0]).astype(bf16))
    for j in range(_NFB):
        w = gelu((jnp.dot(w, wh_ref[0, j], preferred_element_type=jnp.float32)
                  + bh_ref[0, j]).astype(bf16))
    nbr = nbr_ref[0, 0]
    iot = lax.broadcasted_iota(jnp.int32, (_E, _AP), 1)
    oh = (iot == nbr).astype(bf16)
    yj = jnp.dot(oh, y_ref[...], preferred_element_type=jnp.float32)
    prod = yj * w
    parts = [prod[k * _CA:(k + 1) * _CA, :] for k in range(_NBH)]
    while len(parts) > 1:
        parts = [parts[j] + parts[j + 1] for j in range(0, len(parts), 2)]
    yagg = parts[0]
    t = gelu(jnp.dot(yagg, f2w_ref[0], preferred_element_type=jnp.float32)
             + f2b_ref[0])
    v = jnp.dot(t, dw_ref[0], preferred_element_type=jnp.float32) + db_ref[0]
    xn = xall_ref[b, pl.program_id(2)] + v
    xall_ref[b, pl.program_id(2)] = xn
    xo_ref[0, 0] = xn


def _full(shape):
    return pl.BlockSpec(shape, lambda *_: tuple(0 for _ in shape))


def kernel(atomic_numbers, positions, cell, cell_offset, neighbors,
           neighbor_mask, atom_mask, emb, filt_Win, filt_bin, filt_Wh,
           filt_bh, in2f_W, in2f_b, f2out_W, f2out_b, dense_W, dense_b):
    f32, bf16 = jnp.float32, jnp.bfloat16
    pada = _AP - _A
    az = jnp.pad(atomic_numbers, ((0, 0), (0, pada))).astype(jnp.int32)[..., None]
    pos = jnp.pad(positions, ((0, 0), (0, pada), (0, 0)))
    phi = pos.astype(bf16)
    plo = (pos - phi.astype(f32)).astype(bf16)
    ptabT = jnp.concatenate(
        [phi.transpose(0, 2, 1), jnp.zeros((_B, 1, _AP), bf16),
         plo.transpose(0, 2, 1), jnp.zeros((_B, 1, _AP), bf16)],
        axis=1)
    nbr = jnp.pad(neighbors, ((0, 0), (0, pada), (0, 0)))
    nbr_km = (nbr.reshape(_B, _NC, _CA, _NBH).transpose(0, 1, 3, 2)
              .reshape(_B, _NC, _E).astype(jnp.int32))
    nbr_k = nbr_km.reshape(_B, _NC, _E, 1)
    nbr_kT = nbr_km.reshape(_B, _NC, 1, _E)
    ehi16 = emb.astype(bf16)
    ehi = jnp.zeros((128, _D), bf16).at[:_MAXZ].set(ehi16)
    elo = jnp.zeros((128, _D), bf16).at[:_MAXZ].set(
        (emb - ehi16.astype(f32)).astype(bf16))
    winp = jnp.pad(filt_Win, ((0, 0), (0, _NGP - _NG), (0, 0))).astype(bf16)
    wh16 = filt_Wh.astype(bf16)

    fijT, cmaskT = pl.pallas_call(
        _fij_kernel, grid=(_B, _NC),
        in_specs=[pl.BlockSpec((1, 1, 1, _E), lambda b, c: (b, c, 0, 0)),
                  pl.BlockSpec((1, 8, _AP), lambda b, c: (b, 0, 0)),
                  pl.BlockSpec((1, 8, _CA), lambda b, c: (b, 0, c))],
        out_specs=[pl.BlockSpec((1, 1, _NGP, _E), lambda b, c: (b, c, 0, 0)),
                   pl.BlockSpec((1, 1, 1, _E), lambda b, c: (b, c, 0, 0))],
        out_shape=[jax.ShapeDtypeStruct((_B, _NC, _NGP, _E), bf16),
                   jax.ShapeDtypeStruct((_B, _NC, 1, _E), jnp.int32)],
    )(nbr_kT, ptabT, ptabT)
    nbr_eff = cmaskT.reshape(_B, _NC, _E, 1)

    xf = pl.pallas_call(
        _mega_kernel, grid=(_NI, _B, _NC),
        in_specs=[
            pl.BlockSpec((1, _AP, 1), lambda i, b, c: (b, 0, 0)),
            pl.BlockSpec((128, _D), lambda i, b, c: (0, 0)),
            pl.BlockSpec((128, _D), lambda i, b, c: (0, 0)),
            pl.BlockSpec((1, 1, _NGP, _E), lambda i, b, c: (b, c, 0, 0)),
            pl.BlockSpec((1, 1, _E, 1), lambda i, b, c: (b, c, 0, 0)),
            pl.BlockSpec((1, _D, _D), lambda i, b, c: (i, 0, 0)),
            pl.BlockSpec((1, 1, _D), lambda i, b, c: (i, 0, 0)),
            pl.BlockSpec((1, _NGP, _D), lambda i, b, c: (i, 0, 0)),
            pl.BlockSpec((1, 1, _D), lambda i, b, c: (i, 0, 0)),
            pl.BlockSpec((1, _NFB, _D, _D), lambda i, b, c: (i, 0, 0, 0)),
            pl.BlockSpec((1, _NFB, 1, _D), lambda i, b, c: (i, 0, 0, 0)),
            pl.BlockSpec((1, _D, _D), lambda i, b, c: (i, 0, 0)),
            pl.BlockSpec((1, 1, _D), lambda i, b, c: (i, 0, 0)),
            pl.BlockSpec((1, _D, _D), lambda i, b, c: (i, 0, 0)),
            pl.BlockSpec((1, 1, _D), lambda i, b, c: (i, 0, 0)),
        ],
        out_specs=pl.BlockSpec((1, 1, _CA, _D), lambda i, b, c: (b, c, 0, 0)),
        out_shape=jax.ShapeDtypeStruct((_B, _NC, _CA, _D), f32),
        scratch_shapes=[pltpu.VMEM((_B, _NC, _CA, _D), f32),
                        pltpu.VMEM((_AP, _D), bf16)],
        compiler_params=pltpu.CompilerParams(
            dimension_semantics=("arbitrary", "arbitrary", "arbitrary")),
    )(az, ehi, elo, fijT, nbr_eff,
      in2f_W, in2f_b.reshape(_NI, 1, _D),
      winp, filt_bin.reshape(_NI, 1, _D),
      wh16, filt_bh.reshape(_NI, _NFB, 1, _D),
      f2out_W, f2out_b.reshape(_NI, 1, _D),
      dense_W, dense_b.reshape(_NI, 1, _D))

    return xf.reshape(_B, _AP, _D)[:, :_A, :]


# speedup vs baseline: 1.4682x; 1.0011x over previous
import jax
import jax.numpy as jnp
from jax import lax
from jax.experimental import pallas as pl
from jax.experimental.pallas import tpu as pltpu

_B, _A, _NBH = 10, 1000, 32
_D = 128
_NG = 25
_NI = 3
_NFB = 3
_CUTOFF = 5.0
_MAXZ = 100

_AP = 1024
_CA = 512
_NC = _AP // _CA
_E = _CA * _NBH
_NGP = 32


def _fij_kernel(nbr_ref, p_ref, pc_ref, fij_ref, c_ref):
    bf16 = jnp.bfloat16
    nbrT = nbr_ref[0, 0]
    iot = lax.broadcasted_iota(jnp.int32, (_AP, _E), 0)
    ohT = (iot == nbrT).astype(bf16)
    dallT = jnp.dot(p_ref[0], ohT, preferred_element_type=jnp.float32)
    pcT = pc_ref[0].astype(jnp.float32)
    dallT = dallT - jnp.concatenate([pcT] * _NBH, axis=1)
    dv = dallT[0:3, :] + dallT[4:7, :]
    r2 = (dv[0:1, :] * dv[0:1, :] + dv[1:2, :] * dv[1:2, :]
          + dv[2:3, :] * dv[2:3, :])
    r = jnp.sqrt(r2)
    width = _CUTOFF / (_NG - 1)
    coeff = -0.5 / (width * width)
    offs = (lax.broadcasted_iota(jnp.int32, (_NGP, _E), 0)
            .astype(jnp.float32) * width)
    fij_ref[0, 0] = jnp.exp(coeff * (r - offs) ** 2).astype(bf16)
    c_ref[0, 0] = jnp.where(r <= _CUTOFF, nbrT, -1)


def _mega_kernel(az_ref, ehi_ref, elo_ref, fij_ref, nbr_ref,
                 iw_ref, ib_ref, win_ref, bin_ref, wh_ref, bh_ref,
                 f2w_ref, f2b_ref, dw_ref, db_ref,
                 xo_ref, xall_ref, y_ref):
    i = pl.program_id(1)
    c = pl.program_id(2)
    gelu = jax.nn.gelu
    bf16 = jnp.bfloat16

    @pl.when(jnp.logical_and(i == 0, c == 0))
    def _init_x():
        azi = az_ref[0]
        ziot = lax.broadcasted_iota(jnp.int32, (_AP, 128), 1)
        ohz = (ziot == azi).astype(bf16)
        xe = jnp.dot(ohz, ehi_ref[...], preferred_element_type=jnp.float32)
        xe = xe + jnp.dot(ohz, elo_ref[...], preferred_element_type=jnp.float32)
        for cc in range(_NC):
            xall_ref[cc] = xe[cc * _CA:(cc + 1) * _CA]

    @pl.when(c == 0)
    def _compute_y():
        for cc in range(_NC):
            yc = jnp.dot(xall_ref[cc], iw_ref[0],
                         preferred_element_type=jnp.float32)
            y_ref[cc * _CA:(cc + 1) * _CA] = (yc + ib_ref[0]).astype(bf16)

    fijT = fij_ref[0, 0]
    w = gelu((lax.dot_general(fijT, win_ref[0], (((0,), (0,)), ((), ())),
                              preferred_element_type=jnp.float32)
              + bin_ref[0]).astype(bf16))
    for j in range(_NFB):
        w = gelu((jnp.dot(w, wh_ref[0, j], preferred_element_type=jnp.float32)
                  + bh_ref[0, j]).astype(bf16))
    nbr = nbr_ref[0, 0]
    iot = lax.broadcasted_iota(jnp.int32, (_E, _AP), 1)
    oh = (iot == nbr).astype(bf16)
    yj = jnp.dot(oh, y_ref[...], preferred_element_type=jnp.float32)
    prod = yj * w
    parts = [prod[k * _CA:(k + 1) * _CA, :] for k in range(_NBH)]
    while len(parts) > 1:
        parts = [parts[j] + parts[j + 1] for j in range(0, len(parts), 2)]
    yagg = parts[0]
    t = gelu(jnp.dot(yagg, f2w_ref[0], preferred_element_type=jnp.float32)
             + f2b_ref[0])
    v = jnp.dot(t, dw_ref[0], preferred_element_type=jnp.float32) + db_ref[0]
    xn = xall_ref[c] + v
    xall_ref[c] = xn
    xo_ref[0, 0] = xn


def _full(shape):
    return pl.BlockSpec(shape, lambda *_: tuple(0 for _ in shape))


def kernel(atomic_numbers, positions, cell, cell_offset, neighbors,
           neighbor_mask, atom_mask, emb, filt_Win, filt_bin, filt_Wh,
           filt_bh, in2f_W, in2f_b, f2out_W, f2out_b, dense_W, dense_b):
    f32, bf16 = jnp.float32, jnp.bfloat16
    pada = _AP - _A
    az = jnp.pad(atomic_numbers, ((0, 0), (0, pada))).astype(jnp.int32)[..., None]
    pos = jnp.pad(positions, ((0, 0), (0, pada), (0, 0)))
    phi = pos.astype(bf16)
    plo = (pos - phi.astype(f32)).astype(bf16)
    ptabT = jnp.concatenate(
        [phi.transpose(0, 2, 1), jnp.zeros((_B, 1, _AP), bf16),
         plo.transpose(0, 2, 1), jnp.zeros((_B, 1, _AP), bf16)],
        axis=1)
    nbr = jnp.pad(neighbors, ((0, 0), (0, pada), (0, 0)))
    nbr_km = (nbr.reshape(_B, _NC, _CA, _NBH).transpose(0, 1, 3, 2)
              .reshape(_B, _NC, _E).astype(jnp.int32))
    nbr_k = nbr_km.reshape(_B, _NC, _E, 1)
    nbr_kT = nbr_km.reshape(_B, _NC, 1, _E)
    ehi16 = emb.astype(bf16)
    ehi = jnp.zeros((128, _D), bf16).at[:_MAXZ].set(ehi16)
    elo = jnp.zeros((128, _D), bf16).at[:_MAXZ].set(
        (emb - ehi16.astype(f32)).astype(bf16))
    winp = jnp.pad(filt_Win, ((0, 0), (0, _NGP - _NG), (0, 0))).astype(bf16)
    wh16 = filt_Wh.astype(bf16)

    fijT, cmaskT = pl.pallas_call(
        _fij_kernel, grid=(_B, _NC),
        in_specs=[pl.BlockSpec((1, 1, 1, _E), lambda b, c: (b, c, 0, 0)),
                  pl.BlockSpec((1, 8, _AP), lambda b, c: (b, 0, 0)),
                  pl.BlockSpec((1, 8, _CA), lambda b, c: (b, 0, c))],
        out_specs=[pl.BlockSpec((1, 1, _NGP, _E), lambda b, c: (b, c, 0, 0)),
                   pl.BlockSpec((1, 1, 1, _E), lambda b, c: (b, c, 0, 0))],
        out_shape=[jax.ShapeDtypeStruct((_B, _NC, _NGP, _E), bf16),
                   jax.ShapeDtypeStruct((_B, _NC, 1, _E), jnp.int32)],
    )(nbr_kT, ptabT, ptabT)
    nbr_eff = cmaskT.reshape(_B, _NC, _E, 1)

    xf = pl.pallas_call(
        _mega_kernel, grid=(_B, _NI, _NC),
        in_specs=[
            pl.BlockSpec((1, _AP, 1), lambda b, i, c: (b, 0, 0)),
            pl.BlockSpec((128, _D), lambda b, i, c: (0, 0)),
            pl.BlockSpec((128, _D), lambda b, i, c: (0, 0)),
            pl.BlockSpec((1, 1, _NGP, _E), lambda b, i, c: (b, c, 0, 0)),
            pl.BlockSpec((1, 1, _E, 1), lambda b, i, c: (b, c, 0, 0)),
            pl.BlockSpec((1, _D, _D), lambda b, i, c: (i, 0, 0)),
            pl.BlockSpec((1, 1, _D), lambda b, i, c: (i, 0, 0)),
            pl.BlockSpec((1, _NGP, _D), lambda b, i, c: (i, 0, 0)),
            pl.BlockSpec((1, 1, _D), lambda b, i, c: (i, 0, 0)),
            pl.BlockSpec((1, _NFB, _D, _D), lambda b, i, c: (i, 0, 0, 0)),
            pl.BlockSpec((1, _NFB, 1, _D), lambda b, i, c: (i, 0, 0, 0)),
            pl.BlockSpec((1, _D, _D), lambda b, i, c: (i, 0, 0)),
            pl.BlockSpec((1, 1, _D), lambda b, i, c: (i, 0, 0)),
            pl.BlockSpec((1, _D, _D), lambda b, i, c: (i, 0, 0)),
            pl.BlockSpec((1, 1, _D), lambda b, i, c: (i, 0, 0)),
        ],
        out_specs=pl.BlockSpec((1, 1, _CA, _D), lambda b, i, c: (b, c, 0, 0)),
        out_shape=jax.ShapeDtypeStruct((_B, _NC, _CA, _D), f32),
        scratch_shapes=[pltpu.VMEM((_NC, _CA, _D), f32),
                        pltpu.VMEM((_AP, _D), bf16)],
        compiler_params=pltpu.CompilerParams(
            dimension_semantics=("parallel", "arbitrary", "arbitrary")),
    )(az, ehi, elo, fijT, nbr_eff,
      in2f_W, in2f_b.reshape(_NI, 1, _D),
      winp, filt_bin.reshape(_NI, 1, _D),
      wh16, filt_bh.reshape(_NI, _NFB, 1, _D),
      f2out_W, f2out_b.reshape(_NI, 1, _D),
      dense_W, dense_b.reshape(_NI, 1, _D))

    return xf.reshape(_B, _AP, _D)[:, :_A, :]
